# baseline (speedup 1.0000x reference)
import functools

import numpy as np
import jax
import jax.numpy as jnp
from jax import lax
from jax.experimental import pallas as pl
from jax.experimental.pallas import tpu as pltpu
from jax.experimental.pallas import tpu_sc as plsc

N_RAYS = 16384
N_SAMPLES = 64
NEAR = 2.0
FAR = 6.0

NUM_SC_CORES = 2
NUM_SC_SUBCORES = 16
NUM_WORKERS = NUM_SC_CORES * NUM_SC_SUBCORES
RPW = N_RAYS // NUM_WORKERS

R1 = 1024
R2 = 1024

_T64 = np.linspace(0.0, 1.0, N_SAMPLES, dtype=np.float32)
_ZVALS = (NEAR * (1.0 - _T64) + FAR * _T64).astype(np.float32)

_EREP = np.zeros((128, 384), dtype=np.float32)
for _m in range(128):
    _EREP[_m, 3 * _m:3 * _m + 3] = 1.0

_T3 = np.zeros((3, 384), dtype=np.float32)
for _m in range(128):
    for _c in range(3):
        _T3[_c, 3 * _m + _c] = 1.0


def _tc1_body(oT_ref, dT_ref, w1T_ref, b1_ref, w2T_ref, b2_ref, w3_ref,
              out_ref):
    oT = oT_ref[...]
    dT = dT_ref[...]
    W1T = w1T_ref[...]
    b1c = b1_ref[...]
    W2T = w2T_ref[...]
    b2c = b2_ref[...]
    w3c = w3_ref[...]

    aT = jnp.broadcast_to(b1c, (128, R1))
    bbT = jnp.zeros((128, R1), jnp.float32)
    for c in range(3):
        aT = aT + W1T[:, c:c + 1] * oT[c:c + 1, :]
        aT = aT + W1T[:, 3 + c:4 + c] * dT[c:c + 1, :]
        bbT = bbT + W1T[:, c:c + 1] * dT[c:c + 1, :]

    W2Tb = W2T.astype(jnp.bfloat16)
    outs = []
    for s in range(N_SAMPLES):
        x = jnp.maximum(aT + float(_ZVALS[s]) * bbT, 0.0)
        h2 = jnp.dot(W2Tb, x.astype(jnp.bfloat16),
                     preferred_element_type=jnp.float32)
        h2 = jnp.maximum(h2 + b2c, 0.0)
        w_s = jnp.sum(h2 * w3c, axis=0, keepdims=True)
        outs.append(w_s)
    out_ref[...] = jnp.concatenate(outs, axis=0).T


def _tc1(oT, dT, W1T, b1c, W2T, b2c, w3c):
    n = oT.shape[1]
    grid = (n // R1,)
    return pl.pallas_call(
        _tc1_body,
        grid=grid,
        in_specs=[
            pl.BlockSpec((3, R1), lambda b: (0, b)),
            pl.BlockSpec((3, R1), lambda b: (0, b)),
            pl.BlockSpec((128, 6), lambda b: (0, 0)),
            pl.BlockSpec((128, 1), lambda b: (0, 0)),
            pl.BlockSpec((64, 128), lambda b: (0, 0)),
            pl.BlockSpec((64, 1), lambda b: (0, 0)),
            pl.BlockSpec((64, 1), lambda b: (0, 0)),
        ],
        out_specs=pl.BlockSpec((R1, 64), lambda b: (b, 0)),
        out_shape=jax.ShapeDtypeStruct((n, 64), jnp.float32),
    )(oT, dT, W1T, b1c, W2T, b2c, w3c)


def _sc_body(rpw, w_hbm, bins_hbm, u_hbm, zv_hbm, out_hbm,
             w_v, out_v, bins_v, u_v, zv_v, cdf_v, hist_v, hist2_v,
             cdf2_v, histb_v, hist2b_v):
    wid = lax.axis_index("s") * NUM_SC_CORES + lax.axis_index("c")
    base = wid * rpw

    pltpu.sync_copy(w_hbm.at[pl.ds(base * 64, rpw * 64)], w_v)
    pltpu.sync_copy(bins_hbm, bins_v)
    pltpu.sync_copy(u_hbm, u_v)
    pltpu.sync_copy(zv_hbm, zv_v)

    iota = lax.iota(jnp.int32, 16)
    ones_i = jnp.ones((16,), jnp.int32)
    zeros_i = jnp.zeros((16,), jnp.int32)
    uu = [u_v[pl.ds(16 * i, 16)] for i in range(4)]
    zvv = [zv_v[pl.ds(16 * i, 16)] for i in range(4)]
    iota16 = [iota + 16 * i for i in range(4)]
    iotaf = [x.astype(jnp.float32) for x in iota16]

    def one_ray(r, cdf_v, hist_v, hist2_v):
        wb = r * 64
        wrow = [w_v[pl.ds(wb + 16 * i, 16)] for i in range(4)]
        mx = jnp.max(jnp.maximum(jnp.maximum(wrow[0], wrow[1]),
                                 jnp.maximum(wrow[2], wrow[3])))
        e = [jnp.exp(x - mx) for x in wrow]
        raw = [plsc.cumsum(x) for x in e]
        t0 = raw[0][15]
        c2s = t0 + raw[1][15]
        c3s = c2s + raw[2][15]
        S = c3s + raw[3][15]
        e_first = raw[0][0]
        ecum62 = raw[3][14] + c3s
        cs = 1e-5 * S
        D = (ecum62 - e_first) + 62.0 * cs
        Dv = jnp.zeros((16,), jnp.float32) + D
        carr = [0.0, t0, c2s, c3s]
        for i in range(5):
            hist_v[pl.ds(16 * i, 16)] = zeros_i
            hist2_v[pl.ds(16 * i, 16)] = zeros_i
        for i in range(4):
            cdfi = ((raw[i] + carr[i]) - e_first + iotaf[i] * cs) / Dv
            cdf_v[pl.ds(16 * i, 16)] = cdfi
            x63 = cdfi * 63.0
            gi = x63.astype(jnp.int32)
            gi = gi + jnp.where(gi.astype(jnp.float32) < x63, 1, 0)
            mask = (iota < 15) if i == 3 else None
            plsc.addupdate_scatter(hist_v, [gi], ones_i, mask=mask)
        hraw = [plsc.cumsum(hist_v[pl.ds(16 * i, 16)]) for i in range(4)]
        h0 = hraw[0][15]
        h01 = h0 + hraw[1][15]
        h012 = h01 + hraw[2][15]
        icar = [0, h0, h01, h012]
        rbase = jnp.full((16,), r * 128, jnp.int32)
        for i in range(4):
            inds = hraw[i] + icar[i]
            below = inds - 1
            above = jnp.minimum(inds, 62)
            clo = plsc.load_gather(cdf_v, [below])
            chi = plsc.load_gather(cdf_v, [above])
            blo = plsc.load_gather(bins_v, [below])
            bhi = plsc.load_gather(bins_v, [above])
            den = chi - clo
            den = jnp.where(den < 1e-5, 1.0, den)
            t = (uu[i] - clo) / den
            z = blo + t * (bhi - blo)
            mf = (z - NEAR) * (63.0 / (FAR - NEAR))
            mi = mf.astype(jnp.int32) + 1
            mi = jnp.minimum(jnp.maximum(mi, 0), 64)
            plsc.store_scatter(out_v, [rbase + iota16[i] + mi], z)
            plsc.addupdate_scatter(hist2_v, [mi], ones_i)
        g2raw = [plsc.cumsum(hist2_v[pl.ds(16 * i, 16)]) for i in range(4)]
        g0 = g2raw[0][15]
        g01 = g0 + g2raw[1][15]
        g012 = g01 + g2raw[2][15]
        gcar = [0, g0, g01, g012]
        for i in range(4):
            pos = rbase + iota16[i] + g2raw[i] + gcar[i]
            plsc.store_scatter(out_v, [pos], zvv[i])

    def body(r, carry):
        one_ray(2 * r, cdf_v, hist_v, hist2_v)
        one_ray(2 * r + 1, cdf2_v, histb_v, hist2b_v)
        return carry

    lax.fori_loop(0, rpw // 2, body, 0)
    pltpu.sync_copy(out_v, out_hbm.at[pl.ds(base * 128, rpw * 128)])


def _sc_sample(w_flat, bins_a, u_a, zv_a):
    n_rays = w_flat.shape[0] // 64
    rpw = n_rays // NUM_WORKERS
    mesh = plsc.VectorSubcoreMesh(core_axis_name="c", subcore_axis_name="s")
    kern = functools.partial(
        pl.kernel,
        out_type=jax.ShapeDtypeStruct((n_rays * 128,), jnp.float32),
        mesh=mesh,
        scratch_types=[
            pltpu.VMEM((rpw * 64,), jnp.float32),
            pltpu.VMEM((rpw * 128,), jnp.float32),
            pltpu.VMEM((64,), jnp.float32),
            pltpu.VMEM((64,), jnp.float32),
            pltpu.VMEM((64,), jnp.float32),
            pltpu.VMEM((64,), jnp.float32),
            pltpu.VMEM((80,), jnp.int32),
            pltpu.VMEM((80,), jnp.int32),
            pltpu.VMEM((64,), jnp.float32),
            pltpu.VMEM((80,), jnp.int32),
            pltpu.VMEM((80,), jnp.int32),
        ],
        compiler_params=pltpu.CompilerParams(needs_layout_passes=False),
    )(functools.partial(_sc_body, rpw))
    return kern(w_flat, bins_a, u_a, zv_a)


def _tc2_body(z_ref, ot_ref, dt_ref, e_ref, out_ref):
    zr = jnp.dot(z_ref[...], e_ref[...],
                 preferred_element_type=jnp.float32)
    out_ref[...] = ot_ref[...] + dt_ref[...] * zr


def _tc2(z_all, o_t, d_t, erep):
    grid = (N_RAYS // R2,)
    return pl.pallas_call(
        _tc2_body,
        grid=grid,
        in_specs=[
            pl.BlockSpec((R2, 128), lambda b: (b, 0)),
            pl.BlockSpec((R2, 384), lambda b: (b, 0)),
            pl.BlockSpec((R2, 384), lambda b: (b, 0)),
            pl.BlockSpec((128, 384), lambda b: (0, 0)),
        ],
        out_specs=pl.BlockSpec((R2, 384), lambda b: (b, 0)),
        out_shape=jax.ShapeDtypeStruct((N_RAYS, 384), jnp.float32),
    )(z_all, o_t, d_t, erep)


def kernel(static_repr, dynamic_repr, ray_origins, ray_directions,
           W1, b1, W2, b2, W3, b3):
    del static_repr, dynamic_repr, b3
    f32 = jnp.float32

    t64 = jnp.linspace(0.0, 1.0, N_SAMPLES, dtype=f32)
    zv = NEAR * (1.0 - t64) + FAR * t64
    bins = 0.5 * (zv[1:] + zv[:-1])
    bins_a = jnp.concatenate([bins, jnp.full((1,), FAR, f32)])
    u_a = jnp.linspace(0.0, 1.0, N_SAMPLES, dtype=f32)

    oT = ray_origins.T.astype(f32)
    dT = ray_directions.T.astype(f32)
    w1t = W1.T.astype(f32)
    b1c = b1.reshape(128, 1).astype(f32)
    w2t = W2.T.astype(f32)
    b2c = b2.reshape(64, 1).astype(f32)
    w3c = W3.reshape(64, 1).astype(f32)

    nchunks = 1
    cn = N_RAYS // nchunks
    z_chunks = []
    for ci in range(nchunks):
        sl = slice(ci * cn, (ci + 1) * cn)
        w2d = _tc1(oT[:, sl], dT[:, sl], w1t, b1c, w2t, b2c, w3c)
        z_chunks.append(_sc_sample(w2d.reshape(-1), bins_a, u_a, zv))
    z_all = jnp.concatenate(z_chunks).reshape(N_RAYS, 128)

    pts2 = (ray_origins[:, None, :].astype(f32)
            + ray_directions[:, None, :].astype(f32) * z_all[:, :, None])
    dirs_exp = jnp.broadcast_to(ray_directions[:, None, :].astype(f32),
                                (N_RAYS, 128, 3))
    return (pts2, dirs_exp, z_all)


# speedup vs baseline: 1.0052x; 1.0052x over previous
import functools

import numpy as np
import jax
import jax.numpy as jnp
from jax import lax
from jax.experimental import pallas as pl
from jax.experimental.pallas import tpu as pltpu
from jax.experimental.pallas import tpu_sc as plsc

N_RAYS = 16384
N_SAMPLES = 64
NEAR = 2.0
FAR = 6.0

NUM_SC_CORES = 2
NUM_SC_SUBCORES = 16
NUM_WORKERS = NUM_SC_CORES * NUM_SC_SUBCORES
RPW = N_RAYS // NUM_WORKERS

R1 = 1024
R2 = 1024

_T64 = np.linspace(0.0, 1.0, N_SAMPLES, dtype=np.float32)
_ZVALS = (NEAR * (1.0 - _T64) + FAR * _T64).astype(np.float32)

_EREP = np.zeros((128, 384), dtype=np.float32)
for _m in range(128):
    _EREP[_m, 3 * _m:3 * _m + 3] = 1.0

_T3 = np.zeros((3, 384), dtype=np.float32)
for _m in range(128):
    for _c in range(3):
        _T3[_c, 3 * _m + _c] = 1.0


def _tc1_body(oT_ref, dT_ref, w1T_ref, b1_ref, w2T_ref, b2_ref, w3_ref,
              out_ref):
    oT = oT_ref[...]
    dT = dT_ref[...]
    W1T = w1T_ref[...]
    b1c = b1_ref[...]
    W2T = w2T_ref[...]
    b2c = b2_ref[...]
    w3c = w3_ref[...]

    aT = jnp.broadcast_to(b1c, (128, R1))
    bbT = jnp.zeros((128, R1), jnp.float32)
    for c in range(3):
        aT = aT + W1T[:, c:c + 1] * oT[c:c + 1, :]
        aT = aT + W1T[:, 3 + c:4 + c] * dT[c:c + 1, :]
        bbT = bbT + W1T[:, c:c + 1] * dT[c:c + 1, :]

    outs = []
    for s in range(N_SAMPLES):
        x = jnp.maximum(aT + float(_ZVALS[s]) * bbT, 0.0)
        h2 = jnp.dot(W2T, x, preferred_element_type=jnp.float32)
        h2 = jnp.maximum(h2 + b2c, 0.0)
        w_s = jnp.sum(h2 * w3c, axis=0, keepdims=True)
        outs.append(w_s)
    out_ref[...] = jnp.concatenate(outs, axis=0).T


def _tc1(oT, dT, W1T, b1c, W2T, b2c, w3c):
    n = oT.shape[1]
    grid = (n // R1,)
    return pl.pallas_call(
        _tc1_body,
        grid=grid,
        in_specs=[
            pl.BlockSpec((3, R1), lambda b: (0, b)),
            pl.BlockSpec((3, R1), lambda b: (0, b)),
            pl.BlockSpec((128, 6), lambda b: (0, 0)),
            pl.BlockSpec((128, 1), lambda b: (0, 0)),
            pl.BlockSpec((64, 128), lambda b: (0, 0)),
            pl.BlockSpec((64, 1), lambda b: (0, 0)),
            pl.BlockSpec((64, 1), lambda b: (0, 0)),
        ],
        out_specs=pl.BlockSpec((R1, 64), lambda b: (b, 0)),
        out_shape=jax.ShapeDtypeStruct((n, 64), jnp.float32),
    )(oT, dT, W1T, b1c, W2T, b2c, w3c)


def _sc_body(rpw, w_hbm, bins_hbm, u_hbm, zv_hbm, out_hbm,
             w_v, out_v, bins_v, u_v, zv_v, cdf_v, hist_v, hist2_v,
             cdf2_v, histb_v, hist2b_v):
    wid = lax.axis_index("s") * NUM_SC_CORES + lax.axis_index("c")
    base = wid * rpw

    pltpu.sync_copy(w_hbm.at[pl.ds(base * 64, rpw * 64)], w_v)
    pltpu.sync_copy(bins_hbm, bins_v)
    pltpu.sync_copy(u_hbm, u_v)
    pltpu.sync_copy(zv_hbm, zv_v)

    iota = lax.iota(jnp.int32, 16)
    ones_i = jnp.ones((16,), jnp.int32)
    zeros_i = jnp.zeros((16,), jnp.int32)
    uu = [u_v[pl.ds(16 * i, 16)] for i in range(4)]
    zvv = [zv_v[pl.ds(16 * i, 16)] for i in range(4)]
    iota16 = [iota + 16 * i for i in range(4)]
    iotaf = [x.astype(jnp.float32) for x in iota16]

    def one_ray(r, cdf_v, hist_v, hist2_v):
        wb = r * 64
        wrow = [w_v[pl.ds(wb + 16 * i, 16)] for i in range(4)]
        mx = jnp.max(jnp.maximum(jnp.maximum(wrow[0], wrow[1]),
                                 jnp.maximum(wrow[2], wrow[3])))
        e = [jnp.exp(x - mx) for x in wrow]
        raw = [plsc.cumsum(x) for x in e]
        t0 = raw[0][15]
        c2s = t0 + raw[1][15]
        c3s = c2s + raw[2][15]
        S = c3s + raw[3][15]
        e_first = raw[0][0]
        ecum62 = raw[3][14] + c3s
        cs = 1e-5 * S
        D = (ecum62 - e_first) + 62.0 * cs
        Dv = jnp.zeros((16,), jnp.float32) + D
        carr = [0.0, t0, c2s, c3s]
        for i in range(5):
            hist_v[pl.ds(16 * i, 16)] = zeros_i
            hist2_v[pl.ds(16 * i, 16)] = zeros_i
        for i in range(4):
            cdfi = ((raw[i] + carr[i]) - e_first + iotaf[i] * cs) / Dv
            cdf_v[pl.ds(16 * i, 16)] = cdfi
            x63 = cdfi * 63.0
            gi = x63.astype(jnp.int32)
            gi = gi + jnp.where(gi.astype(jnp.float32) < x63, 1, 0)
            mask = (iota < 15) if i == 3 else None
            plsc.addupdate_scatter(hist_v, [gi], ones_i, mask=mask)
        hraw = [plsc.cumsum(hist_v[pl.ds(16 * i, 16)]) for i in range(4)]
        h0 = hraw[0][15]
        h01 = h0 + hraw[1][15]
        h012 = h01 + hraw[2][15]
        icar = [0, h0, h01, h012]
        rbase = jnp.full((16,), r * 128, jnp.int32)
        for i in range(4):
            inds = hraw[i] + icar[i]
            below = inds - 1
            above = jnp.minimum(inds, 62)
            clo = plsc.load_gather(cdf_v, [below])
            chi = plsc.load_gather(cdf_v, [above])
            blo = plsc.load_gather(bins_v, [below])
            bhi = plsc.load_gather(bins_v, [above])
            den = chi - clo
            den = jnp.where(den < 1e-5, 1.0, den)
            t = (uu[i] - clo) / den
            z = blo + t * (bhi - blo)
            mf = (z - NEAR) * (63.0 / (FAR - NEAR))
            mi = mf.astype(jnp.int32) + 1
            mi = jnp.minimum(jnp.maximum(mi, 0), 64)
            plsc.store_scatter(out_v, [rbase + iota16[i] + mi], z)
            plsc.addupdate_scatter(hist2_v, [mi], ones_i)
        g2raw = [plsc.cumsum(hist2_v[pl.ds(16 * i, 16)]) for i in range(4)]
        g0 = g2raw[0][15]
        g01 = g0 + g2raw[1][15]
        g012 = g01 + g2raw[2][15]
        gcar = [0, g0, g01, g012]
        for i in range(4):
            pos = rbase + iota16[i] + g2raw[i] + gcar[i]
            plsc.store_scatter(out_v, [pos], zvv[i])

    def body(r, carry):
        one_ray(2 * r, cdf_v, hist_v, hist2_v)
        one_ray(2 * r + 1, cdf2_v, histb_v, hist2b_v)
        return carry

    lax.fori_loop(0, rpw // 2, body, 0)
    pltpu.sync_copy(out_v, out_hbm.at[pl.ds(base * 128, rpw * 128)])


def _sc_sample(w_flat, bins_a, u_a, zv_a):
    n_rays = w_flat.shape[0] // 64
    rpw = n_rays // NUM_WORKERS
    mesh = plsc.VectorSubcoreMesh(core_axis_name="c", subcore_axis_name="s")
    kern = functools.partial(
        pl.kernel,
        out_type=jax.ShapeDtypeStruct((n_rays * 128,), jnp.float32),
        mesh=mesh,
        scratch_types=[
            pltpu.VMEM((rpw * 64,), jnp.float32),
            pltpu.VMEM((rpw * 128,), jnp.float32),
            pltpu.VMEM((64,), jnp.float32),
            pltpu.VMEM((64,), jnp.float32),
            pltpu.VMEM((64,), jnp.float32),
            pltpu.VMEM((64,), jnp.float32),
            pltpu.VMEM((80,), jnp.int32),
            pltpu.VMEM((80,), jnp.int32),
            pltpu.VMEM((64,), jnp.float32),
            pltpu.VMEM((80,), jnp.int32),
            pltpu.VMEM((80,), jnp.int32),
        ],
        compiler_params=pltpu.CompilerParams(needs_layout_passes=False),
    )(functools.partial(_sc_body, rpw))
    return kern(w_flat, bins_a, u_a, zv_a)


def _tc2_body(z_ref, ot_ref, dt_ref, e_ref, out_ref):
    zr = jnp.dot(z_ref[...], e_ref[...],
                 preferred_element_type=jnp.float32)
    out_ref[...] = ot_ref[...] + dt_ref[...] * zr


def _tc2(z_all, o_t, d_t, erep):
    grid = (N_RAYS // R2,)
    return pl.pallas_call(
        _tc2_body,
        grid=grid,
        in_specs=[
            pl.BlockSpec((R2, 128), lambda b: (b, 0)),
            pl.BlockSpec((R2, 384), lambda b: (b, 0)),
            pl.BlockSpec((R2, 384), lambda b: (b, 0)),
            pl.BlockSpec((128, 384), lambda b: (0, 0)),
        ],
        out_specs=pl.BlockSpec((R2, 384), lambda b: (b, 0)),
        out_shape=jax.ShapeDtypeStruct((N_RAYS, 384), jnp.float32),
    )(z_all, o_t, d_t, erep)


def kernel(static_repr, dynamic_repr, ray_origins, ray_directions,
           W1, b1, W2, b2, W3, b3):
    del static_repr, dynamic_repr, b3
    f32 = jnp.float32

    t64 = jnp.linspace(0.0, 1.0, N_SAMPLES, dtype=f32)
    zv = NEAR * (1.0 - t64) + FAR * t64
    bins = 0.5 * (zv[1:] + zv[:-1])
    bins_a = jnp.concatenate([bins, jnp.full((1,), FAR, f32)])
    u_a = jnp.linspace(0.0, 1.0, N_SAMPLES, dtype=f32)

    oT = ray_origins.T.astype(f32)
    dT = ray_directions.T.astype(f32)
    w1t = W1.T.astype(f32)
    b1c = b1.reshape(128, 1).astype(f32)
    w2t = W2.T.astype(f32)
    b2c = b2.reshape(64, 1).astype(f32)
    w3c = W3.reshape(64, 1).astype(f32)

    nchunks = 1
    cn = N_RAYS // nchunks
    z_chunks = []
    for ci in range(nchunks):
        sl = slice(ci * cn, (ci + 1) * cn)
        w2d = _tc1(oT[:, sl], dT[:, sl], w1t, b1c, w2t, b2c, w3c)
        z_chunks.append(_sc_sample(w2d.reshape(-1), bins_a, u_a, zv))
    z_all = jnp.concatenate(z_chunks).reshape(N_RAYS, 128)

    pts2 = (ray_origins[:, None, :].astype(f32)
            + ray_directions[:, None, :].astype(f32) * z_all[:, :, None])
    dirs_exp = jnp.broadcast_to(ray_directions[:, None, :].astype(f32),
                                (N_RAYS, 128, 3))
    return (pts2, dirs_exp, z_all)
